# baseline (speedup 1.0000x reference)
import functools

import jax
import jax.numpy as jnp
from jax import lax
from jax.experimental import pallas as pl
from jax.experimental.pallas import tpu as pltpu
from jax.experimental.pallas import tpu_sc as plsc

SPELL_LENGTH = 9
HIDDEN_SIZE = 128
N_SLICES = 2


@functools.lru_cache(maxsize=None)
def _build_gather(Bs):
    info = plsc.get_sparse_core_info()
    L = info.num_lanes
    NW = info.num_cores * info.num_subcores
    b_per_w = Bs // NW
    rows_w = b_per_w * SPELL_LENGTH
    CH = max(c for c in range(1, 129) if rows_w % c == 0)
    n_chunks = rows_w // CH
    NBUF = min(6, n_chunks)
    n_vregs = rows_w // L

    mesh = plsc.VectorSubcoreMesh(core_axis_name="c", subcore_axis_name="s")

    @functools.partial(
        pl.kernel,
        mesh=mesh,
        out_type=jax.ShapeDtypeStruct((Bs * SPELL_LENGTH, HIDDEN_SIZE),
                                      jnp.float32),
        scratch_types=[
            pltpu.VMEM((b_per_w,), jnp.int32),
            pltpu.VMEM((rows_w,), jnp.int32),
            pltpu.VMEM((NBUF, CH, HIDDEN_SIZE), jnp.float32),
            pltpu.SemaphoreType.DMA,
            pltpu.SemaphoreType.DMA,
        ],
        compiler_params=pltpu.CompilerParams(needs_layout_passes=False),
    )
    def gather_kernel(table_hbm, rs_hbm, out_hbm, rs_v, eidx_v, rows_v,
                      sem_g, sem_s):
        wid = lax.axis_index("s") * info.num_cores + lax.axis_index("c")
        pltpu.sync_copy(rs_hbm.at[pl.ds(wid * b_per_w, b_per_w)], rs_v)

        lane = lax.broadcasted_iota(jnp.int32, (L,), 0)
        for v in range(n_vregs):
            p = lane + (L * v)
            s = lax.shift_right_logical(p * 7282, 16)
            j = p - s * SPELL_LENGTH
            r = plsc.load_gather(rs_v, [s])
            eidx_v[pl.ds(L * v, L)] = r * SPELL_LENGTH + j

        base = wid * rows_w

        def gather(c, slot):
            return pltpu.async_copy(
                table_hbm.at[eidx_v.at[pl.ds(c * CH, CH)]],
                rows_v.at[slot], sem_g)

        def put(c, slot):
            return pltpu.async_copy(
                rows_v.at[slot], out_hbm.at[pl.ds(base + c * CH, CH)], sem_s)

        ahead = max(1, NBUF - 2)
        g = [None] * n_chunks
        s_ = [None] * n_chunks
        put_waited = [False] * n_chunks
        for c in range(min(ahead, n_chunks)):
            g[c] = gather(c, c % NBUF)
        for c in range(n_chunks):
            g[c].wait()
            s_[c] = put(c, c % NBUF)
            nxt = c + ahead
            if nxt < n_chunks:
                prev = nxt - NBUF
                if prev >= 0:
                    s_[prev].wait()
                    put_waited[prev] = True
                g[nxt] = gather(nxt, nxt % NBUF)
        for c in range(n_chunks):
            if not put_waited[c]:
                s_[c].wait()

    return gather_kernel


@functools.lru_cache(maxsize=None)
def _build_unflatten(B, Bs, k, aliased):
    S = min(1024, Bs // 2)
    grid = Bs // S
    blk0 = k * (Bs // S)

    def body(*refs):
        in_ref, out_ref = refs[-2], refs[-1]
        for i in range(S):
            out_ref[i] = in_ref[pl.ds(SPELL_LENGTH * i, SPELL_LENGTH), :]

    in_specs = [pl.BlockSpec((S * SPELL_LENGTH, HIDDEN_SIZE),
                             lambda g: (g, 0))]
    if aliased:
        in_specs = [pl.BlockSpec(memory_space=pl.ANY)] + in_specs
    return pl.pallas_call(
        body,
        grid=(grid,),
        in_specs=in_specs,
        out_specs=pl.BlockSpec((S, SPELL_LENGTH, HIDDEN_SIZE),
                               lambda g: (g + blk0, 0, 0)),
        out_shape=jax.ShapeDtypeStruct((B, SPELL_LENGTH, HIDDEN_SIZE),
                                       jnp.float32),
        input_output_aliases={0: 0} if aliased else {},
    )


def kernel(rs_tensor, embedding_relation):
    B = rs_tensor.shape[0]
    Bs = B // N_SLICES
    rs = rs_tensor.astype(jnp.int32)
    gather_sc = _build_gather(Bs)
    flats = [gather_sc(embedding_relation, rs[k * Bs:(k + 1) * Bs])
             for k in range(N_SLICES)]
    out = _build_unflatten(B, Bs, 0, False)(flats[0])
    for k in range(1, N_SLICES):
        out = _build_unflatten(B, Bs, k, True)(out, flats[k])
    return out


# speedup vs baseline: 1.0089x; 1.0089x over previous
import functools

import jax
import jax.numpy as jnp
from jax import lax
from jax.experimental import pallas as pl
from jax.experimental.pallas import tpu as pltpu
from jax.experimental.pallas import tpu_sc as plsc

SPELL_LENGTH = 9
HIDDEN_SIZE = 128
N_SLICES = 2


@functools.lru_cache(maxsize=None)
def _build_gather(Bs):
    info = plsc.get_sparse_core_info()
    L = info.num_lanes
    NW = info.num_cores * info.num_subcores
    b_per_w = Bs // NW
    rows_w = b_per_w * SPELL_LENGTH
    CH = max(c for c in range(1, 129) if rows_w % c == 0)
    n_chunks = rows_w // CH
    NBUF = min(6, n_chunks)
    n_vregs = rows_w // L

    mesh = plsc.VectorSubcoreMesh(core_axis_name="c", subcore_axis_name="s")

    @functools.partial(
        pl.kernel,
        mesh=mesh,
        out_type=jax.ShapeDtypeStruct((Bs * SPELL_LENGTH, HIDDEN_SIZE),
                                      jnp.float32),
        scratch_types=[
            pltpu.VMEM((b_per_w,), jnp.int32),
            pltpu.VMEM((rows_w,), jnp.int32),
            pltpu.VMEM((NBUF, CH, HIDDEN_SIZE), jnp.float32),
            pltpu.SemaphoreType.DMA,
            pltpu.SemaphoreType.DMA,
        ],
        compiler_params=pltpu.CompilerParams(needs_layout_passes=False),
    )
    def gather_kernel(table_hbm, rs_hbm, out_hbm, rs_v, eidx_v, rows_v,
                      sem_g, sem_s):
        wid = lax.axis_index("s") * info.num_cores + lax.axis_index("c")
        pltpu.sync_copy(rs_hbm.at[pl.ds(wid * b_per_w, b_per_w)], rs_v)

        lane = lax.broadcasted_iota(jnp.int32, (L,), 0)
        for v in range(n_vregs):
            p = lane + (L * v)
            s = lax.shift_right_logical(p * 7282, 16)
            j = p - s * SPELL_LENGTH
            r = plsc.load_gather(rs_v, [s])
            eidx_v[pl.ds(L * v, L)] = r * SPELL_LENGTH + j

        base = wid * rows_w

        def gather(c, slot):
            return pltpu.async_copy(
                table_hbm.at[eidx_v.at[pl.ds(c * CH, CH)]],
                rows_v.at[slot], sem_g)

        def put(c, slot):
            return pltpu.async_copy(
                rows_v.at[slot], out_hbm.at[pl.ds(base + c * CH, CH)], sem_s)

        ahead = max(1, NBUF - 2)
        g = [None] * n_chunks
        s_ = [None] * n_chunks
        put_waited = [False] * n_chunks
        for c in range(min(ahead, n_chunks)):
            g[c] = gather(c, c % NBUF)
        for c in range(n_chunks):
            g[c].wait()
            s_[c] = put(c, c % NBUF)
            nxt = c + ahead
            if nxt < n_chunks:
                prev = nxt - NBUF
                if prev >= 0:
                    s_[prev].wait()
                    put_waited[prev] = True
                g[nxt] = gather(nxt, nxt % NBUF)
        for c in range(n_chunks):
            if not put_waited[c]:
                s_[c].wait()

    return gather_kernel


@functools.lru_cache(maxsize=None)
def _build_unflatten(B, Bs, k, aliased):
    S = min(1024, Bs // 2)
    grid = Bs // S
    blk0 = k * (Bs // S)

    def body(*refs):
        in_ref, out_ref = refs[0], refs[-1]
        for i in range(S):
            out_ref[i] = in_ref[pl.ds(SPELL_LENGTH * i, SPELL_LENGTH), :]

    in_specs = [pl.BlockSpec((S * SPELL_LENGTH, HIDDEN_SIZE),
                             lambda g: (g, 0))]
    if aliased:
        in_specs = in_specs + [pl.BlockSpec(memory_space=pl.ANY)]
    return pl.pallas_call(
        body,
        grid=(grid,),
        in_specs=in_specs,
        out_specs=pl.BlockSpec((S, SPELL_LENGTH, HIDDEN_SIZE),
                               lambda g: (g + blk0, 0, 0)),
        out_shape=jax.ShapeDtypeStruct((B, SPELL_LENGTH, HIDDEN_SIZE),
                                       jnp.float32),
        input_output_aliases={1: 0} if aliased else {},
    )


def kernel(rs_tensor, embedding_relation):
    B = rs_tensor.shape[0]
    Bs = B // N_SLICES
    rs = rs_tensor.astype(jnp.int32)
    gather_sc = _build_gather(Bs)
    flats = [gather_sc(embedding_relation, rs[k * Bs:(k + 1) * Bs])
             for k in range(N_SLICES)]
    out = _build_unflatten(B, Bs, 0, False)(flats[0])
    for k in range(1, N_SLICES):
        out = _build_unflatten(B, Bs, k, True)(flats[k], out)
    return out
